# baseline (device time: 15138 ns/iter reference)
import jax
import jax.numpy as jnp
from jax import lax
from jax.experimental import pallas as pl
from jax.experimental.pallas import tpu as pltpu

N_DEV = 8


def kernel(x, W, labels):
    t, d = x.shape
    _, v_per = W.shape

    def body(x_ref, w_ref, lab_ref, out_ref,
             w_buf, stats_ref, gather_ref, w_sems, send_sems, recv_sems):
        my_pos = lax.axis_index("i")

        half = d // 2
        cp0 = pltpu.make_async_copy(
            w_ref.at[pl.ds(0, half), :], w_buf.at[0], w_sems.at[0])
        cp1 = pltpu.make_async_copy(
            w_ref.at[pl.ds(half, half), :], w_buf.at[1], w_sems.at[1])
        cp0.start()
        cp1.start()

        barrier_sem = pltpu.get_barrier_semaphore()
        for off in range(1, N_DEV):
            nbr = (my_pos + off) % N_DEV
            pl.semaphore_signal(barrier_sem, inc=1, device_id=(nbr,),
                                device_id_type=pl.DeviceIdType.MESH)

        cp0.wait()
        acc = jnp.dot(x_ref[:, :half], w_buf[0],
                      preferred_element_type=jnp.float32)
        cp1.wait()
        logits = acc + jnp.dot(x_ref[:, half:], w_buf[1],
                               preferred_element_type=jnp.float32)
        s_loc = jnp.sum(jnp.exp(logits), axis=1)

        local_idx = lab_ref[:] - my_pos * v_per
        cols = lax.broadcasted_iota(jnp.int32, (t, v_per), 1)
        c_loc = jnp.sum(jnp.where(cols == local_idx[:, None], logits, 0.0),
                        axis=1)

        stats_ref[:, :] = jnp.concatenate(
            [s_loc[None, :], c_loc[None, :]], axis=0)

        pl.semaphore_wait(barrier_sem, N_DEV - 1)

        rdmas = []
        for off in range(1, N_DEV):
            tgt = (my_pos + off) % N_DEV
            rdma = pltpu.make_async_remote_copy(
                src_ref=stats_ref,
                dst_ref=gather_ref.at[off - 1],
                send_sem=send_sems.at[off - 1],
                recv_sem=recv_sems.at[off - 1],
                device_id=(tgt,),
                device_id_type=pl.DeviceIdType.MESH,
            )
            rdma.start()
            rdmas.append(rdma)
        for rdma in rdmas:
            rdma.wait_recv()

        g = gather_ref[:, :, :]
        s_g = s_loc + jnp.sum(g[:, 0, :], axis=0)
        c_g = c_loc + jnp.sum(g[:, 1, :], axis=0)
        out_ref[:] = jnp.log(s_g) - c_g

        for rdma in rdmas:
            rdma.wait_send()

    return pl.pallas_call(
        body,
        out_shape=jax.ShapeDtypeStruct((t,), jnp.float32),
        in_specs=[
            pl.BlockSpec(memory_space=pltpu.VMEM),
            pl.BlockSpec(memory_space=pl.ANY),
            pl.BlockSpec(memory_space=pltpu.VMEM),
        ],
        out_specs=pl.BlockSpec(memory_space=pltpu.VMEM),
        scratch_shapes=[
            pltpu.VMEM((2, d // 2, v_per), jnp.float32),
            pltpu.VMEM((2, t), jnp.float32),
            pltpu.VMEM((N_DEV - 1, 2, t), jnp.float32),
            pltpu.SemaphoreType.DMA((2,)),
            pltpu.SemaphoreType.DMA((N_DEV - 1,)),
            pltpu.SemaphoreType.DMA((N_DEV - 1,)),
        ],
        compiler_params=pltpu.CompilerParams(collective_id=0),
    )(x, W, labels)


# device time: 14753 ns/iter; 1.0261x vs baseline; 1.0261x over previous
import jax
import jax.numpy as jnp
from jax import lax
from jax.experimental import pallas as pl
from jax.experimental.pallas import tpu as pltpu

N_DEV = 8


def kernel(x, W, labels):
    t, d = x.shape
    _, v_per = W.shape

    def body(x_ref, w_ref, lab_ref, out_ref,
             stats_ref, gather_ref, send_sems, recv_sems):
        my_pos = lax.axis_index("i")

        barrier_sem = pltpu.get_barrier_semaphore()
        for off in range(1, N_DEV):
            nbr = (my_pos + off) % N_DEV
            pl.semaphore_signal(barrier_sem, inc=1, device_id=(nbr,),
                                device_id_type=pl.DeviceIdType.MESH)

        logits = jnp.dot(x_ref[:, :], w_ref[:, :],
                         preferred_element_type=jnp.float32)

        local_idx = lab_ref[:] - my_pos * v_per
        cols = lax.broadcasted_iota(jnp.int32, (t, v_per), 1)
        masked = jnp.where(cols == local_idx[:, None], logits, 0.0)

        ones = jnp.ones((v_per, 1), jnp.float32)
        s_loc = jnp.dot(jnp.exp(logits), ones,
                        preferred_element_type=jnp.float32)[:, 0]
        c_loc = jnp.dot(masked, ones,
                        preferred_element_type=jnp.float32)[:, 0]

        stats_ref[:, :] = jnp.concatenate(
            [s_loc[None, :], c_loc[None, :]], axis=0)

        pl.semaphore_wait(barrier_sem, N_DEV - 1)

        rdmas = []
        for off in range(1, N_DEV):
            tgt = (my_pos + off) % N_DEV
            rdma = pltpu.make_async_remote_copy(
                src_ref=stats_ref,
                dst_ref=gather_ref.at[off - 1],
                send_sem=send_sems.at[off - 1],
                recv_sem=recv_sems.at[off - 1],
                device_id=(tgt,),
                device_id_type=pl.DeviceIdType.MESH,
            )
            rdma.start()
            rdmas.append(rdma)
        for rdma in rdmas:
            rdma.wait_recv()

        g = gather_ref[:, :, :]
        s_g = s_loc + jnp.sum(g[:, 0, :], axis=0)
        c_g = c_loc + jnp.sum(g[:, 1, :], axis=0)
        out_ref[:] = jnp.log(s_g) - c_g

        for rdma in rdmas:
            rdma.wait_send()

    return pl.pallas_call(
        body,
        out_shape=jax.ShapeDtypeStruct((t,), jnp.float32),
        in_specs=[
            pl.BlockSpec(memory_space=pltpu.VMEM),
            pl.BlockSpec(memory_space=pltpu.VMEM),
            pl.BlockSpec(memory_space=pltpu.VMEM),
        ],
        out_specs=pl.BlockSpec(memory_space=pltpu.VMEM),
        scratch_shapes=[
            pltpu.VMEM((2, t), jnp.float32),
            pltpu.VMEM((N_DEV - 1, 2, t), jnp.float32),
            pltpu.SemaphoreType.DMA((N_DEV - 1,)),
            pltpu.SemaphoreType.DMA((N_DEV - 1,)),
        ],
        compiler_params=pltpu.CompilerParams(collective_id=0),
    )(x, W, labels)


# device time: 13831 ns/iter; 1.0945x vs baseline; 1.0667x over previous
import jax
import jax.numpy as jnp
from jax import lax
from jax.experimental import pallas as pl
from jax.experimental.pallas import tpu as pltpu

N_DEV = 8


def kernel(x, W, labels):
    t, d = x.shape
    _, v_per = W.shape

    def body(x_ref, w_ref, lab_ref, out_ref,
             stats_ref, gather_ref, send_sems, recv_sems):
        my_pos = lax.axis_index("i")

        barrier_sem = pltpu.get_barrier_semaphore()
        for off in range(1, N_DEV):
            nbr = (my_pos + off) % N_DEV
            pl.semaphore_signal(barrier_sem, inc=1, device_id=(nbr,),
                                device_id_type=pl.DeviceIdType.MESH)

        logits = jnp.dot(x_ref[:, :], w_ref[:, :],
                         preferred_element_type=jnp.float32)
        s_loc = jnp.sum(jnp.exp(logits), axis=1)

        local_idx = lab_ref[:] - my_pos * v_per
        cols = lax.broadcasted_iota(jnp.int32, (t, v_per), 1)
        c_loc = jnp.sum(jnp.where(cols == local_idx[:, None], logits, 0.0),
                        axis=1)

        stats_ref[:, :] = jnp.concatenate(
            [s_loc[None, :], c_loc[None, :]], axis=0)

        pl.semaphore_wait(barrier_sem, N_DEV - 1)

        rdmas = []
        for off in range(1, N_DEV):
            tgt = (my_pos + off) % N_DEV
            rdma = pltpu.make_async_remote_copy(
                src_ref=stats_ref,
                dst_ref=gather_ref.at[off - 1],
                send_sem=send_sems.at[off - 1],
                recv_sem=recv_sems.at[off - 1],
                device_id=(tgt,),
                device_id_type=pl.DeviceIdType.MESH,
            )
            rdma.start()
            rdmas.append(rdma)
        for rdma in rdmas:
            rdma.wait_recv()

        g = gather_ref[:, :, :]
        s_g = s_loc + jnp.sum(g[:, 0, :], axis=0)
        c_g = c_loc + jnp.sum(g[:, 1, :], axis=0)
        out_ref[:] = jnp.log(s_g) - c_g

        for rdma in rdmas:
            rdma.wait_send()

    return pl.pallas_call(
        body,
        out_shape=jax.ShapeDtypeStruct((t,), jnp.float32),
        in_specs=[
            pl.BlockSpec(memory_space=pltpu.VMEM),
            pl.BlockSpec(memory_space=pltpu.VMEM),
            pl.BlockSpec(memory_space=pltpu.VMEM),
        ],
        out_specs=pl.BlockSpec(memory_space=pltpu.VMEM),
        scratch_shapes=[
            pltpu.VMEM((2, t), jnp.float32),
            pltpu.VMEM((N_DEV - 1, 2, t), jnp.float32),
            pltpu.SemaphoreType.DMA((N_DEV - 1,)),
            pltpu.SemaphoreType.DMA((N_DEV - 1,)),
        ],
        compiler_params=pltpu.CompilerParams(collective_id=0),
    )(x, W, labels)
